# baseline (speedup 1.0000x reference)
import functools
import math

import jax
import jax.numpy as jnp
from jax.experimental import pallas as pl
from jax.experimental.pallas import tpu as pltpu

EPS = 1e-5
KS = 3
IMGS_PER_STEP = 2


def _conv_stats_kernel(x_ref, w_ref, mask_ref, y_ref, stats_ref,
                       p0_ref, p1_ref, *, cin, hw, width):
    for img in range(IMGS_PER_STEP):
        patch_ref = p0_ref if img % 2 == 0 else p1_ref
        for ky in range(KS):
            for kx in range(KS):
                tap = ky * KS + kx
                t = x_ref[img, :, pl.ds(ky * width + kx, hw)]
                if kx == 0:
                    t = t * mask_ref[0:1, :]
                elif kx == KS - 1:
                    t = t * mask_ref[1:2, :]
                patch_ref[pl.ds(tap * cin, cin), :] = t

        y = jnp.dot(w_ref[...], patch_ref[...],
                    preferred_element_type=jnp.float32)
        y_ref[img] = y.astype(jnp.bfloat16)
        stats_ref[img, :, 0:1] = jnp.sum(y, axis=1, keepdims=True)
        stats_ref[img, :, 1:2] = jnp.sum(y * y, axis=1, keepdims=True)


def _bn_relu_kernel(y_ref, scale_ref, bias_ref, o_ref):
    y = y_ref[...].astype(jnp.float32)
    o_ref[...] = jnp.maximum(y * scale_ref[...] + bias_ref[...], 0.0)


def kernel(x, weight, gamma, beta):
    n, cin, h, width = x.shape
    cout = weight.shape[0]
    hw = h * width
    flat = hw + 2 * (width + 1)

    xb = x.astype(jnp.bfloat16)
    xf = jnp.pad(xb, ((0, 0), (0, 0), (1, 1), (0, 0)))
    xf = xf.reshape(n, cin, (h + 2) * width)
    xf = jnp.pad(xf, ((0, 0), (0, 0), (1, 1)))

    w_mat = weight.transpose(0, 2, 3, 1).reshape(cout, KS * KS * cin)
    w_mat = w_mat.astype(jnp.bfloat16)

    col = jnp.arange(hw, dtype=jnp.int32) % width
    mask = jnp.stack([col != 0, col != width - 1]).astype(jnp.bfloat16)

    m = IMGS_PER_STEP
    kern = functools.partial(_conv_stats_kernel, cin=cin, hw=hw, width=width)
    y_flat, stats = pl.pallas_call(
        kern,
        grid=(n // m,),
        in_specs=[pl.BlockSpec((m, cin, flat), lambda i: (i, 0, 0)),
                  pl.BlockSpec((cout, KS * KS * cin), lambda i: (0, 0)),
                  pl.BlockSpec((2, hw), lambda i: (0, 0))],
        out_specs=[pl.BlockSpec((m, cout, hw), lambda i: (i, 0, 0)),
                   pl.BlockSpec((m, cout, 2), lambda i: (i, 0, 0))],
        out_shape=[jax.ShapeDtypeStruct((n, cout, hw), jnp.bfloat16),
                   jax.ShapeDtypeStruct((n, cout, 2), jnp.float32)],
        scratch_shapes=[pltpu.VMEM((KS * KS * cin, hw), jnp.bfloat16),
                        pltpu.VMEM((KS * KS * cin, hw), jnp.bfloat16)],
        compiler_params=pltpu.CompilerParams(
            dimension_semantics=("parallel",)),
    )(xf, w_mat, mask)

    cnt = n * hw
    g32 = gamma.astype(jnp.float32)
    mean = jnp.sum(stats[:, :, 0], axis=0) / cnt
    var = jnp.maximum(jnp.sum(stats[:, :, 1], axis=0) / cnt - mean * mean, 0.0)
    inv = jax.lax.rsqrt(var + EPS)
    scale = (g32 * inv).reshape(cout, 1)
    bias = (beta.astype(jnp.float32) - mean * g32 * inv).reshape(cout, 1)

    out_flat = pl.pallas_call(
        _bn_relu_kernel,
        grid=(n // m,),
        in_specs=[pl.BlockSpec((m, cout, hw), lambda i: (i, 0, 0)),
                  pl.BlockSpec((cout, 1), lambda i: (0, 0)),
                  pl.BlockSpec((cout, 1), lambda i: (0, 0))],
        out_specs=pl.BlockSpec((m, cout, hw), lambda i: (i, 0, 0)),
        out_shape=jax.ShapeDtypeStruct((n, cout, hw), x.dtype),
        compiler_params=pltpu.CompilerParams(
            dimension_semantics=("parallel",)),
    )(y_flat, scale, bias)

    return out_flat.reshape(n, cout, h, width)


# speedup vs baseline: 1.1091x; 1.1091x over previous
import functools
import math

import jax
import jax.numpy as jnp
from jax.experimental import pallas as pl
from jax.experimental.pallas import tpu as pltpu

EPS = 1e-5
KS = 3
IMGS_PER_STEP = 2


def _conv_stats_kernel(x_ref, w_ref, mask_ref, y_ref, stats_ref,
                       xx0_ref, xx1_ref, p0_ref, p1_ref, *, cin, hw, width):
    g = width + 1
    for img in range(IMGS_PER_STEP):
        xx_ref = xx0_ref if img % 2 == 0 else xx1_ref
        patch_ref = p0_ref if img % 2 == 0 else p1_ref
        xx_ref[:, pl.ds(0, g)] = jnp.zeros((cin, g), jnp.bfloat16)
        xx_ref[:, pl.ds(g + hw, g)] = jnp.zeros((cin, g), jnp.bfloat16)
        xx_ref[:, pl.ds(g, hw)] = x_ref[img]
        for ky in range(KS):
            for kx in range(KS):
                tap = ky * KS + kx
                t = xx_ref[:, pl.ds(ky * width + kx, hw)]
                if kx == 0:
                    t = t * mask_ref[0:1, :]
                elif kx == KS - 1:
                    t = t * mask_ref[1:2, :]
                patch_ref[pl.ds(tap * cin, cin), :] = t

        y = jnp.dot(w_ref[...], patch_ref[...],
                    preferred_element_type=jnp.float32)
        y_ref[img] = y.astype(jnp.bfloat16)
        stats_ref[img, :, 0:1] = jnp.sum(y, axis=1, keepdims=True)
        stats_ref[img, :, 1:2] = jnp.sum(y * y, axis=1, keepdims=True)


def _bn_relu_kernel(y_ref, scale_ref, bias_ref, o_ref):
    y = y_ref[...].astype(jnp.float32)
    o_ref[...] = jnp.maximum(y * scale_ref[...] + bias_ref[...], 0.0)


def kernel(x, weight, gamma, beta):
    n, cin, h, width = x.shape
    cout = weight.shape[0]
    hw = h * width
    flat = hw + 2 * (width + 1)

    xf = x.reshape(n, cin, hw).astype(jnp.bfloat16)

    w_mat = weight.transpose(0, 2, 3, 1).reshape(cout, KS * KS * cin)
    w_mat = w_mat.astype(jnp.bfloat16)

    col = jnp.arange(hw, dtype=jnp.int32) % width
    mask = jnp.stack([col != 0, col != width - 1]).astype(jnp.bfloat16)

    m = IMGS_PER_STEP
    kern = functools.partial(_conv_stats_kernel, cin=cin, hw=hw, width=width)
    y_flat, stats = pl.pallas_call(
        kern,
        grid=(n // m,),
        in_specs=[pl.BlockSpec((m, cin, hw), lambda i: (i, 0, 0)),
                  pl.BlockSpec((cout, KS * KS * cin), lambda i: (0, 0)),
                  pl.BlockSpec((2, hw), lambda i: (0, 0))],
        out_specs=[pl.BlockSpec((m, cout, hw), lambda i: (i, 0, 0)),
                   pl.BlockSpec((m, cout, 2), lambda i: (i, 0, 0))],
        out_shape=[jax.ShapeDtypeStruct((n, cout, hw), jnp.bfloat16),
                   jax.ShapeDtypeStruct((n, cout, 2), jnp.float32)],
        scratch_shapes=[pltpu.VMEM((cin, flat), jnp.bfloat16),
                        pltpu.VMEM((cin, flat), jnp.bfloat16),
                        pltpu.VMEM((KS * KS * cin, hw), jnp.bfloat16),
                        pltpu.VMEM((KS * KS * cin, hw), jnp.bfloat16)],
        compiler_params=pltpu.CompilerParams(
            dimension_semantics=("parallel",)),
    )(xf, w_mat, mask)

    cnt = n * hw
    g32 = gamma.astype(jnp.float32)
    mean = jnp.sum(stats[:, :, 0], axis=0) / cnt
    var = jnp.maximum(jnp.sum(stats[:, :, 1], axis=0) / cnt - mean * mean, 0.0)
    inv = jax.lax.rsqrt(var + EPS)
    scale = (g32 * inv).reshape(cout, 1)
    bias = (beta.astype(jnp.float32) - mean * g32 * inv).reshape(cout, 1)

    out_flat = pl.pallas_call(
        _bn_relu_kernel,
        grid=(n // m,),
        in_specs=[pl.BlockSpec((m, cout, hw), lambda i: (i, 0, 0)),
                  pl.BlockSpec((cout, 1), lambda i: (0, 0)),
                  pl.BlockSpec((cout, 1), lambda i: (0, 0))],
        out_specs=pl.BlockSpec((m, cout, hw), lambda i: (i, 0, 0)),
        out_shape=jax.ShapeDtypeStruct((n, cout, hw), x.dtype),
        compiler_params=pltpu.CompilerParams(
            dimension_semantics=("parallel",)),
    )(y_flat, scale, bias)

    return out_flat.reshape(n, cout, h, width)


# speedup vs baseline: 1.1703x; 1.0552x over previous
import functools
import math

import jax
import jax.numpy as jnp
from jax.experimental import pallas as pl
from jax.experimental.pallas import tpu as pltpu

EPS = 1e-5
KS = 3
IMGS_PER_STEP = 2


def _conv_stats_kernel(x_ref, w_ref, mask_ref, y_ref, stats_ref,
                       xx0_ref, xx1_ref, p0_ref, p1_ref, *, cin, hw, width):
    g = width + 1
    for img in range(IMGS_PER_STEP):
        xx_ref = xx0_ref if img % 2 == 0 else xx1_ref
        patch_ref = p0_ref if img % 2 == 0 else p1_ref
        xx_ref[:, pl.ds(0, g)] = jnp.zeros((cin, g), jnp.bfloat16)
        xx_ref[:, pl.ds(g + hw, g)] = jnp.zeros((cin, g), jnp.bfloat16)
        xx_ref[:, pl.ds(g, hw)] = x_ref[img].astype(jnp.bfloat16)
        for ky in range(KS):
            for kx in range(KS):
                tap = ky * KS + kx
                t = xx_ref[:, pl.ds(ky * width + kx, hw)]
                if kx == 0:
                    t = t * mask_ref[0:1, :]
                elif kx == KS - 1:
                    t = t * mask_ref[1:2, :]
                patch_ref[pl.ds(tap * cin, cin), :] = t

        y = jnp.dot(w_ref[...], patch_ref[...],
                    preferred_element_type=jnp.float32)
        y_ref[img] = y.astype(jnp.bfloat16)
        stats_ref[img, :, 0:1] = jnp.sum(y, axis=1, keepdims=True)
        stats_ref[img, :, 1:2] = jnp.sum(y * y, axis=1, keepdims=True)


def _bn_relu_kernel(y_ref, scale_ref, bias_ref, o_ref):
    y = y_ref[...].astype(jnp.float32)
    o_ref[...] = jnp.maximum(y * scale_ref[...] + bias_ref[...], 0.0)


def kernel(x, weight, gamma, beta):
    n, cin, h, width = x.shape
    cout = weight.shape[0]
    hw = h * width
    flat = hw + 2 * (width + 1)

    xf = x.reshape(n, cin, hw)

    w_mat = weight.transpose(0, 2, 3, 1).reshape(cout, KS * KS * cin)
    w_mat = w_mat.astype(jnp.bfloat16)

    col = jnp.arange(hw, dtype=jnp.int32) % width
    mask = jnp.stack([col != 0, col != width - 1]).astype(jnp.bfloat16)

    m = IMGS_PER_STEP
    kern = functools.partial(_conv_stats_kernel, cin=cin, hw=hw, width=width)
    y_flat, stats = pl.pallas_call(
        kern,
        grid=(n // m,),
        in_specs=[pl.BlockSpec((m, cin, hw), lambda i: (i, 0, 0)),
                  pl.BlockSpec((cout, KS * KS * cin), lambda i: (0, 0)),
                  pl.BlockSpec((2, hw), lambda i: (0, 0))],
        out_specs=[pl.BlockSpec((m, cout, hw), lambda i: (i, 0, 0)),
                   pl.BlockSpec((m, cout, 2), lambda i: (i, 0, 0))],
        out_shape=[jax.ShapeDtypeStruct((n, cout, hw), jnp.bfloat16),
                   jax.ShapeDtypeStruct((n, cout, 2), jnp.float32)],
        scratch_shapes=[pltpu.VMEM((cin, flat), jnp.bfloat16),
                        pltpu.VMEM((cin, flat), jnp.bfloat16),
                        pltpu.VMEM((KS * KS * cin, hw), jnp.bfloat16),
                        pltpu.VMEM((KS * KS * cin, hw), jnp.bfloat16)],
        compiler_params=pltpu.CompilerParams(
            dimension_semantics=("parallel",)),
    )(xf, w_mat, mask)

    cnt = n * hw
    g32 = gamma.astype(jnp.float32)
    mean = jnp.sum(stats[:, :, 0], axis=0) / cnt
    var = jnp.maximum(jnp.sum(stats[:, :, 1], axis=0) / cnt - mean * mean, 0.0)
    inv = jax.lax.rsqrt(var + EPS)
    scale = (g32 * inv).reshape(cout, 1)
    bias = (beta.astype(jnp.float32) - mean * g32 * inv).reshape(cout, 1)

    out_flat = pl.pallas_call(
        _bn_relu_kernel,
        grid=(n // m,),
        in_specs=[pl.BlockSpec((m, cout, hw), lambda i: (i, 0, 0)),
                  pl.BlockSpec((cout, 1), lambda i: (0, 0)),
                  pl.BlockSpec((cout, 1), lambda i: (0, 0))],
        out_specs=pl.BlockSpec((m, cout, hw), lambda i: (i, 0, 0)),
        out_shape=jax.ShapeDtypeStruct((n, cout, hw), x.dtype),
        compiler_params=pltpu.CompilerParams(
            dimension_semantics=("parallel",)),
    )(y_flat, scale, bias)

    return out_flat.reshape(n, cout, h, width)
